# baseline (speedup 1.0000x reference)
import functools

import jax
import jax.numpy as jnp
from jax import lax
from jax.experimental import pallas as pl
from jax.experimental.pallas import tpu as pltpu
from jax.experimental.pallas import tpu_sc as plsc

_NC = 2
_NS = 16
_NW = _NC * _NS
_LANES = 16

_ROWS = 8192
_COLS = 4096
_N = _ROWS * _COLS

_SC_ROWS = 3584
_TC_ROWS = _ROWS - _SC_ROWS
_TC_BLOCK_ROWS = 256

_ROWS_W = _SC_ROWS // _NW
_CROWS = 4
_NBUF = 4
_NCHUNK = _ROWS_W // _CROWS
_NGROUP = _NCHUNK // _NBUF
_UNROLL = 8
_NACC = 4

_mesh = plsc.VectorSubcoreMesh(core_axis_name="c", subcore_axis_name="s")


@functools.partial(
    pl.kernel,
    out_type=jax.ShapeDtypeStruct((_NW, _LANES), jnp.int32),
    mesh=_mesh,
    scratch_types=[
        pltpu.VMEM((_CROWS, _COLS), jnp.float32),
        pltpu.VMEM((_CROWS, _COLS), jnp.float32),
        pltpu.VMEM((_CROWS, _COLS), jnp.float32),
        pltpu.VMEM((_CROWS, _COLS), jnp.float32),
        pltpu.VMEM((_LANES,), jnp.int32),
        pltpu.SemaphoreType.DMA,
        pltpu.SemaphoreType.DMA,
        pltpu.SemaphoreType.DMA,
        pltpu.SemaphoreType.DMA,
    ],
)
def _count_positives(values_hbm, out_hbm, buf0, buf1, buf2, buf3, acc_vmem,
                     sem0, sem1, sem2, sem3):
    wid = lax.axis_index("s") * _NC + lax.axis_index("c")
    base = wid * _ROWS_W
    bufs = (buf0, buf1, buf2, buf3)
    sems = (sem0, sem1, sem2, sem3)

    def dma(chunk_idx, buf, sem):
        src = values_hbm.at[pl.ds(base + chunk_idx * _CROWS, _CROWS), :]
        return pltpu.make_async_copy(src, buf, sem)

    for b in range(_NBUF):
        dma(b, bufs[b], sems[b]).start()

    def reduce_chunk(buf, accs):
        for r in range(_CROWS):
            row = buf.at[r]
            def rbody(j, a, row=row):
                a = list(a)
                off = j * (_LANES * _UNROLL)
                for u in range(_UNROLL):
                    v = row[pl.ds(off + u * _LANES, _LANES)]
                    k = u % _NACC
                    a[k] = a[k] + jnp.where(v > 0.0, 1.0, 0.0)
                return tuple(a)
            accs = lax.fori_loop(0, _COLS // (_LANES * _UNROLL), rbody, accs)
        return accs

    def group_body(g, accs):
        for b in range(_NBUF):
            dma(0, bufs[b], sems[b]).wait()
            accs = reduce_chunk(bufs[b], accs)
            dma(_NBUF * g + b + _NBUF, bufs[b], sems[b]).start()
        return accs

    accs = tuple(jnp.zeros((_LANES,), jnp.float32) for _ in range(_NACC))
    accs = lax.fori_loop(0, _NGROUP - 1, group_body, accs)
    for b in range(_NBUF):
        dma(0, bufs[b], sems[b]).wait()
        accs = reduce_chunk(bufs[b], accs)

    acc = accs[0]
    for k in range(1, _NACC):
        acc = acc + accs[k]
    acc_vmem[...] = acc.astype(jnp.int32)
    pltpu.sync_copy(acc_vmem, out_hbm.at[wid])


def _tc_count_body(xa_ref, xb_ref, o_ref):
    @pl.when(pl.program_id(0) == 0)
    def _init():
        o_ref[0, 0] = jnp.int32(0)
    sa = jnp.sum(jnp.where(xa_ref[...] > 0.0, 1, 0))
    sb = jnp.sum(jnp.where(xb_ref[...] > 0.0, 1, 0))
    o_ref[0, 0] = o_ref[0, 0] + sa + sb


def _entropy_body(parts_ref, tc_ref, o_ref):
    c1 = jnp.sum(parts_ref[...]) + tc_ref[0, 0]
    c1f = c1.astype(jnp.float32)
    c0f = (_N - c1).astype(jnp.float32)
    denom = jnp.float32(_N + 1e-8)
    p0 = c0f / denom
    p1 = c1f / denom
    t0 = jnp.where(p0 > 0, p0 * jnp.log2(p0 + 1e-10), 0.0)
    t1 = jnp.where(p1 > 0, p1 * jnp.log2(p1 + 1e-10), 0.0)
    o_ref[...] = jnp.broadcast_to(-(t0 + t1), (1, 1))


def kernel(values):
    parts = _count_positives(values)
    half_blocks = _TC_ROWS // (2 * _TC_BLOCK_ROWS)
    tc_count = pl.pallas_call(
        _tc_count_body,
        grid=(half_blocks,),
        in_specs=[
            pl.BlockSpec(
                (_TC_BLOCK_ROWS, _COLS),
                lambda i: (i + _SC_ROWS // _TC_BLOCK_ROWS, 0),
            ),
            pl.BlockSpec(
                (_TC_BLOCK_ROWS, _COLS),
                lambda i: (i + _SC_ROWS // _TC_BLOCK_ROWS + half_blocks, 0),
            ),
        ],
        out_specs=pl.BlockSpec(memory_space=pltpu.SMEM),
        out_shape=jax.ShapeDtypeStruct((1, 1), jnp.int32),
    )(values, values)
    out = pl.pallas_call(
        _entropy_body,
        in_specs=[
            pl.BlockSpec((_NW, _LANES), lambda: (0, 0)),
            pl.BlockSpec(memory_space=pltpu.SMEM),
        ],
        out_shape=jax.ShapeDtypeStruct((1, 1), jnp.float32),
    )(parts, tc_count)
    return out[0, 0]


# speedup vs baseline: 1.0082x; 1.0082x over previous
import functools

import jax
import jax.numpy as jnp
from jax import lax
from jax.experimental import pallas as pl
from jax.experimental.pallas import tpu as pltpu
from jax.experimental.pallas import tpu_sc as plsc

_NC = 2
_NS = 16
_NW = _NC * _NS
_LANES = 16

_ROWS = 8192
_COLS = 4096
_N = _ROWS * _COLS

_SC_ROWS = 4096
_TC_ROWS = _ROWS - _SC_ROWS
_TC_BLOCK_ROWS = 512

_ROWS_W = _SC_ROWS // _NW
_CROWS = 4
_NBUF = 4
_NCHUNK = _ROWS_W // _CROWS
_NGROUP = _NCHUNK // _NBUF
_UNROLL = 8
_NACC = 4

_mesh = plsc.VectorSubcoreMesh(core_axis_name="c", subcore_axis_name="s")


@functools.partial(
    pl.kernel,
    out_type=jax.ShapeDtypeStruct((_NW, _LANES), jnp.int32),
    mesh=_mesh,
    scratch_types=[
        pltpu.VMEM((_CROWS, _COLS), jnp.float32),
        pltpu.VMEM((_CROWS, _COLS), jnp.float32),
        pltpu.VMEM((_CROWS, _COLS), jnp.float32),
        pltpu.VMEM((_CROWS, _COLS), jnp.float32),
        pltpu.VMEM((_LANES,), jnp.int32),
        pltpu.SemaphoreType.DMA,
        pltpu.SemaphoreType.DMA,
        pltpu.SemaphoreType.DMA,
        pltpu.SemaphoreType.DMA,
    ],
)
def _count_positives(values_hbm, out_hbm, buf0, buf1, buf2, buf3, acc_vmem,
                     sem0, sem1, sem2, sem3):
    wid = lax.axis_index("s") * _NC + lax.axis_index("c")
    base = wid * _ROWS_W
    bufs = (buf0, buf1, buf2, buf3)
    sems = (sem0, sem1, sem2, sem3)

    def dma(chunk_idx, buf, sem):
        src = values_hbm.at[pl.ds(base + chunk_idx * _CROWS, _CROWS), :]
        return pltpu.make_async_copy(src, buf, sem)

    for b in range(_NBUF):
        dma(b, bufs[b], sems[b]).start()

    def reduce_chunk(buf, accs):
        for r in range(_CROWS):
            row = buf.at[r]
            def rbody(j, a, row=row):
                a = list(a)
                off = j * (_LANES * _UNROLL)
                for u in range(_UNROLL):
                    v = row[pl.ds(off + u * _LANES, _LANES)]
                    k = u % _NACC
                    a[k] = a[k] + jnp.where(v > 0.0, 1.0, 0.0)
                return tuple(a)
            accs = lax.fori_loop(0, _COLS // (_LANES * _UNROLL), rbody, accs)
        return accs

    def group_body(g, accs):
        for b in range(_NBUF):
            dma(0, bufs[b], sems[b]).wait()
            accs = reduce_chunk(bufs[b], accs)
            dma(_NBUF * g + b + _NBUF, bufs[b], sems[b]).start()
        return accs

    accs = tuple(jnp.zeros((_LANES,), jnp.float32) for _ in range(_NACC))
    accs = lax.fori_loop(0, _NGROUP - 1, group_body, accs)
    for b in range(_NBUF):
        dma(0, bufs[b], sems[b]).wait()
        accs = reduce_chunk(bufs[b], accs)

    acc = accs[0]
    for k in range(1, _NACC):
        acc = acc + accs[k]
    acc_vmem[...] = acc.astype(jnp.int32)
    pltpu.sync_copy(acc_vmem, out_hbm.at[wid])


def _tc_count_body(xa_ref, xb_ref, o_ref):
    @pl.when(pl.program_id(0) == 0)
    def _init():
        o_ref[0, 0] = jnp.int32(0)
    sa = jnp.sum(jnp.where(xa_ref[...] > 0.0, 1, 0))
    sb = jnp.sum(jnp.where(xb_ref[...] > 0.0, 1, 0))
    o_ref[0, 0] = o_ref[0, 0] + sa + sb


def _entropy_body(parts_ref, tc_ref, o_ref):
    c1 = jnp.sum(parts_ref[...]) + tc_ref[0, 0]
    c1f = c1.astype(jnp.float32)
    c0f = (_N - c1).astype(jnp.float32)
    denom = jnp.float32(_N + 1e-8)
    p0 = c0f / denom
    p1 = c1f / denom
    t0 = jnp.where(p0 > 0, p0 * jnp.log2(p0 + 1e-10), 0.0)
    t1 = jnp.where(p1 > 0, p1 * jnp.log2(p1 + 1e-10), 0.0)
    o_ref[...] = jnp.broadcast_to(-(t0 + t1), (1, 1))


def kernel(values):
    parts = _count_positives(values)
    half_blocks = _TC_ROWS // (2 * _TC_BLOCK_ROWS)
    tc_count = pl.pallas_call(
        _tc_count_body,
        grid=(half_blocks,),
        in_specs=[
            pl.BlockSpec(
                (_TC_BLOCK_ROWS, _COLS),
                lambda i: (i + _SC_ROWS // _TC_BLOCK_ROWS, 0),
            ),
            pl.BlockSpec(
                (_TC_BLOCK_ROWS, _COLS),
                lambda i: (i + _SC_ROWS // _TC_BLOCK_ROWS + half_blocks, 0),
            ),
        ],
        out_specs=pl.BlockSpec(memory_space=pltpu.SMEM),
        out_shape=jax.ShapeDtypeStruct((1, 1), jnp.int32),
    )(values, values)
    out = pl.pallas_call(
        _entropy_body,
        in_specs=[
            pl.BlockSpec((_NW, _LANES), lambda: (0, 0)),
            pl.BlockSpec(memory_space=pltpu.SMEM),
        ],
        out_shape=jax.ShapeDtypeStruct((1, 1), jnp.float32),
    )(parts, tc_count)
    return out[0, 0]
